# baseline (speedup 1.0000x reference)
import jax
import jax.numpy as jnp
from jax.experimental import pallas as pl

_PATCH = 16
_RATIO = 0.75


def _mask_body(noise_ref, out_ref):
    n = jax.lax.bitcast_convert_type(noise_ref[:], jnp.int32)
    rb, nn = n.shape
    num_mask = int(_RATIO * nn)
    num_rest = nn - num_mask
    kc = 256

    bq = n[:, None, num_mask:]

    nm1 = n - 1
    g = jnp.zeros((rb, num_rest), jnp.int32)
    for k0 in range(0, num_mask, kc):
        cmp = nm1[:, k0:k0 + kc, None] < bq
        g = g + jnp.sum(cmp, axis=1, dtype=jnp.int32)
    nk = n[:, num_mask:, None]
    k_iota = jax.lax.broadcasted_iota(jnp.int32, (1, num_rest, num_rest), 1)
    q_iota = jax.lax.broadcasted_iota(jnp.int32, (1, num_rest, num_rest), 2)
    m = (k_iota < q_iota).astype(jnp.int32)
    g = g + jnp.sum((nk - m) < bq, axis=1, dtype=jnp.int32)

    nw = nn // 16
    gh = g[:, None, :] >> 4
    pw = 1 << (g[:, None, :] & 15)
    w_iota = jax.lax.broadcasted_iota(jnp.int32, (1, nw, 1), 1)
    contrib = jnp.where(gh == w_iota, pw, 0)
    bitmap = jnp.sum(contrib, axis=2)
    b_iota = jax.lax.broadcasted_iota(jnp.int32, (1, 1, 16), 2)
    bits = (bitmap[:, :, None] >> b_iota) & 1
    out_ref[:] = bits.reshape(rb, nn) == 0


def kernel(img, noise):
    num_patches = (img.shape[2] // _PATCH) * (img.shape[3] // _PATCH)
    bsz = noise.shape[0]
    assert noise.shape[1] == num_patches
    keys = noise
    rb = 64
    out = pl.pallas_call(
        _mask_body,
        grid=(bsz // rb,),
        in_specs=[pl.BlockSpec((rb, num_patches), lambda i: (i, 0))],
        out_specs=pl.BlockSpec((rb, num_patches), lambda i: (i, 0)),
        out_shape=jax.ShapeDtypeStruct((bsz, num_patches), jnp.bool_),
    )(keys)
    return out
